# baseline (speedup 1.0000x reference)
import functools

import jax
import jax.numpy as jnp
from jax import lax
from jax.experimental import pallas as pl
from jax.experimental.pallas import tpu as pltpu
from jax.experimental.pallas import tpu_sc as plsc

NUM_E = 1000000
NUM_R = 1000
B = 16384
D = 64

_info = plsc.get_sparse_core_info()
_NC, _NS, _L = _info.num_cores, _info.num_subcores, _info.num_lanes
_NW = _NC * _NS
_BPW = B // _NW
_CHUNK = 128
_NCHUNK = _BPW // _CHUNK

_BK = 8192
_K1_GRID = 32
_LQ = _BK * _K1_GRID
_HI_MAX = (NUM_E - 1) // _BK
_LQ_SHIFT = _LQ.bit_length() - 1


def _k1_body(x0_ref, x1_ref, x2_ref, x3_ref, tlo_ref, thi_ref, o_ref):
  dn = (((0,), (0,)), ((), ()))
  tlo = tlo_ref[...].astype(jnp.bfloat16)
  thi = thi_ref[...].astype(jnp.bfloat16)
  los, his = [], []
  for xr in (x0_ref, x1_ref, x2_ref, x3_ref):
    xb = xr[...].astype(jnp.bfloat16)
    los.append(lax.dot_general(xb, tlo, dn,
                               preferred_element_type=jnp.float32))
    his.append(lax.dot_general(xb, thi, dn,
                               preferred_element_type=jnp.float32))
  lo = jnp.concatenate(los, axis=1)
  hi = jnp.concatenate(his, axis=1)
  lo16 = lax.bitcast_convert_type(lo.astype(jnp.bfloat16), jnp.uint16)
  hi16 = lax.bitcast_convert_type(hi.astype(jnp.bfloat16), jnp.uint16)
  packed = (hi16.astype(jnp.uint32) << 16) | lo16.astype(jnp.uint32)
  o_ref[...] = lax.bitcast_convert_type(packed, jnp.int32)


def _k1_proj(ent_t, tlo, thi):
  def xspec(p):
    return pl.BlockSpec(
        (D, _BK), lambda i, p=p: (0, jnp.minimum(i + p * _K1_GRID, _HI_MAX)))

  return pl.pallas_call(
      _k1_body,
      grid=(_K1_GRID,),
      in_specs=[xspec(0), xspec(1), xspec(2), xspec(3),
                pl.BlockSpec((D, D // 2), lambda i: (0, 0)),
                pl.BlockSpec((D, D // 2), lambda i: (0, 0))],
      out_specs=pl.BlockSpec((_BK, 2 * D), lambda i: (i, 0)),
      out_shape=jax.ShapeDtypeStruct((_LQ, 2 * D), jnp.int32),
  )(ent_t, ent_t, ent_t, ent_t, tlo, thi)


def _k2_body(proj_hbm, rel_hbm, jh_hbm, jt_hbm, jr_hbm, out_hbm,
             jh, jt, jr, h2, t2, r2, ov, hsem, tsem, rsem):
  wid = lax.axis_index("s") * _NC + lax.axis_index("c")
  base = wid * _BPW
  pltpu.sync_copy(jh_hbm.at[wid], jh)
  pltpu.sync_copy(jt_hbm.at[wid], jt)
  pltpu.sync_copy(jr_hbm.at[wid], jr)

  nw = D // 2

  def fire(k, buf):
    return (pltpu.async_copy(proj_hbm.at[jh.at[k]], h2.at[buf], hsem),
            pltpu.async_copy(proj_hbm.at[jt.at[k]], t2.at[buf], tsem),
            pltpu.async_copy(rel_hbm.at[jr.at[k]], r2.at[buf], rsem))

  pend = fire(0, 0)
  for k in range(_NCHUNK):
    nxt = fire(k + 1, (k + 1) % 2) if k + 1 < _NCHUNK else None
    for c in pend:
      c.wait()
    buf = k % 2

    def row(i, carry, buf=buf):
      for c in range(nw // _L):
        sl = pl.ds(c * _L, _L)
        hv = plsc.bitcast(h2[buf, i, sl], jnp.bfloat16)
        tv = plsc.bitcast(t2[buf, i, sl], jnp.bfloat16)
        rv = plsc.bitcast(r2[buf, i, sl], jnp.bfloat16)
        ov[i, pl.ds(c * 2 * _L, 2 * _L)] = hv - tv + rv
      return carry

    lax.fori_loop(0, _CHUNK, row, 0)
    pltpu.sync_copy(ov, out_hbm.at[pl.ds(base + k * _CHUNK, _CHUNK)])
    pend = nxt


_k2_gather = functools.partial(
    pl.kernel,
    mesh=plsc.VectorSubcoreMesh(core_axis_name="c", subcore_axis_name="s"),
    compiler_params=pltpu.CompilerParams(
        use_tc_tiling_on_sc=False, needs_layout_passes=False),
    out_type=jax.ShapeDtypeStruct((B, D), jnp.bfloat16),
    scratch_types=[
        pltpu.VMEM((_NCHUNK, _CHUNK), jnp.int32),
        pltpu.VMEM((_NCHUNK, _CHUNK), jnp.int32),
        pltpu.VMEM((_NCHUNK, _CHUNK), jnp.int32),
        pltpu.VMEM((2, _CHUNK, D // 2), jnp.int32),
        pltpu.VMEM((2, _CHUNK, D // 2), jnp.int32),
        pltpu.VMEM((2, _CHUNK, D // 2), jnp.int32),
        pltpu.VMEM((_CHUNK, D), jnp.bfloat16),
        pltpu.SemaphoreType.DMA,
        pltpu.SemaphoreType.DMA,
        pltpu.SemaphoreType.DMA,
    ],
)(_k2_body)


def kernel(head, relation, tail, ent_emb, rel_emb, transfer):
  head = head.astype(jnp.int32)
  tail = tail.astype(jnp.int32)
  relation = relation.astype(jnp.int32)
  shape3 = (_NW, _NCHUNK, _CHUNK)
  jh = (4 * (head & (_LQ - 1)) + (head >> _LQ_SHIFT)).reshape(shape3)
  jt = (4 * (tail & (_LQ - 1)) + (tail >> _LQ_SHIFT)).reshape(shape3)
  jr = relation.reshape(shape3)
  tlo = transfer[:, 0::2]
  thi = transfer[:, 1::2]
  rel_i32 = lax.bitcast_convert_type(
      rel_emb.astype(jnp.bfloat16).reshape(NUM_R, D // 2, 2), jnp.int32)
  proj = _k1_proj(ent_emb.T, tlo, thi).reshape(4 * _LQ, D // 2)
  out16 = _k2_gather(proj, rel_i32, jh, jt, jr)
  return out16.astype(jnp.float32)
